# Initial kernel scaffold; baseline (speedup 1.0000x reference)
import jax
import jax.numpy as jnp
from jax.experimental import pallas as pl


def kernel(xs, ds):
    raise NotImplementedError("write your pallas kernel here")


# speedup vs baseline: 4.9460x; 4.9460x over previous
import jax
import jax.numpy as jnp
from jax.experimental import pallas as pl

WINDOW_SIZE = 16.0
N_ITER = 8
INV_SIGMA2 = 1.0 / (2.0 * WINDOW_SIZE * WINDOW_SIZE)
EPS = 1e-8


def _stgw_body(ds_ref, xs_ref, out_ref):
    T = xs_ref.shape[1]

    d = ds_ref[0].astype(jnp.float32)

    ii = jax.lax.broadcasted_iota(jnp.int32, (T, T), 0)
    ij = jax.lax.broadcasted_iota(jnp.int32, (T, T), 1)
    tri = (ii <= ij).astype(jnp.float32)
    io = ii.astype(jnp.float32)
    cum = jnp.dot(d, tri, preferred_element_type=jnp.float32)

    total = jnp.maximum(cum[:, T - 1 : T], 1.0)
    centers = (cum - 0.5 * d) * (jnp.float32(T) / total)

    diff = (io + 0.5) - centers
    logits = -(diff * diff) * INV_SIGMA2

    m = jnp.max(logits, axis=1, keepdims=True)
    e = jnp.exp(logits - m)
    r = 1.0 / jnp.sum(e, axis=1, keepdims=True)

    v = jnp.ones((1, T), dtype=jnp.float32)
    for _ in range(N_ITER):
        s = jnp.sum(e * r, axis=0, keepdims=True)
        v = v / (v * s + EPS)
        z = jnp.sum(e * v, axis=1, keepdims=True)
        r = r / (r * z + EPS)

    ev = e * v
    ys = jnp.dot(ev, xs_ref[0], preferred_element_type=jnp.float32)
    out_ref[0] = ys * r


@jax.jit
def kernel(xs, ds):
    B, T, D = xs.shape
    ds3 = ds.reshape(B, 1, T)
    return pl.pallas_call(
        _stgw_body,
        grid=(B,),
        in_specs=[
            pl.BlockSpec((1, 1, T), lambda b: (b, 0, 0)),
            pl.BlockSpec((1, T, D), lambda b: (b, 0, 0)),
        ],
        out_specs=pl.BlockSpec((1, T, D), lambda b: (b, 0, 0)),
        out_shape=jax.ShapeDtypeStruct((B, T, D), jnp.float32),
    )(ds3, xs)


# speedup vs baseline: 5.0258x; 1.0161x over previous
import jax
import jax.numpy as jnp
from jax.experimental import pallas as pl

WINDOW_SIZE = 16.0
N_ITER = 8
INV_SIGMA2 = 1.0 / (2.0 * WINDOW_SIZE * WINDOW_SIZE)
EPS = 1e-8


def _stgw_body(ds_ref, xs_ref, out_ref):
    T = xs_ref.shape[1]

    d = ds_ref[0].astype(jnp.float32)

    ii = jax.lax.broadcasted_iota(jnp.int32, (T, T), 0)
    ij = jax.lax.broadcasted_iota(jnp.int32, (T, T), 1)
    tri = (ii <= ij).astype(jnp.float32)
    io = ii.astype(jnp.float32)
    cum = jnp.dot(d, tri, preferred_element_type=jnp.float32)

    total = jnp.maximum(cum[:, T - 1 : T], 1.0)
    centers = (cum - 0.5 * d) * (jnp.float32(T) / total)

    diff = (io + 0.5) - centers
    logits = -(diff * diff) * INV_SIGMA2

    m = jnp.max(logits, axis=1, keepdims=True)
    e = jnp.exp(logits - m)

    a = jnp.sum(e, axis=1, keepdims=True)
    b = jnp.ones((1, T), dtype=jnp.float32)
    for _ in range(N_ITER):
        r = 1.0 / a
        s = jnp.sum(e * r, axis=0, keepdims=True)
        b = s + EPS * b
        v = 1.0 / b
        z = jnp.sum(e * v, axis=1, keepdims=True)
        a = z + EPS * a

    ev = e * v
    ys = jnp.dot(ev, xs_ref[0], preferred_element_type=jnp.float32)
    out_ref[0] = ys * (1.0 / a)


@jax.jit
def kernel(xs, ds):
    B, T, D = xs.shape
    ds3 = ds.reshape(B, 1, T)
    return pl.pallas_call(
        _stgw_body,
        grid=(B,),
        in_specs=[
            pl.BlockSpec((1, 1, T), lambda b: (b, 0, 0)),
            pl.BlockSpec((1, T, D), lambda b: (b, 0, 0)),
        ],
        out_specs=pl.BlockSpec((1, T, D), lambda b: (b, 0, 0)),
        out_shape=jax.ShapeDtypeStruct((B, T, D), jnp.float32),
    )(ds3, xs)
